# baseline (speedup 1.0000x reference)
import jax
import jax.numpy as jnp
from jax.experimental import pallas as pl
from jax.experimental.pallas import tpu as pltpu

K_NN = 20
N_PTS = 40
BB = 16


def _edge_layer(feats, Wl, bl, n, d_in):
    sq = jnp.sum(feats * feats, axis=-1)
    grams = []
    for b in range(BB):
        fb = feats[b]
        grams.append(jax.lax.dot_general(
            fb, fb, (((1,), (1,)), ((), ())),
            preferred_element_type=jnp.float32))
    G = jnp.stack(grams)
    nd = 2.0 * G - sq[:, :, None] - sq[:, None, :]

    a = nd[:, :, :, None]
    c = nd[:, :, None, :]
    ii = jax.lax.broadcasted_iota(jnp.int32, (n, n), 0)
    jj = jax.lax.broadcasted_iota(jnp.int32, (n, n), 1)
    tie = (ii < jj).astype(jnp.float32)[None, None, :, :]
    cmp = (a > c).astype(jnp.float32) + (a == c).astype(jnp.float32) * tie
    rank = jnp.sum(cmp, axis=2)
    sel = rank < float(K_NN)

    hs = []
    for b in range(BB):
        fc = feats[b]
        ef_c = jnp.broadcast_to(fc[:, None, :], (n, n, d_in))
        ef_d = fc[None, :, :] - fc[:, None, :]
        ef = jnp.concatenate([ef_c, ef_d], axis=-1).reshape(n * n, 2 * d_in)
        h = jnp.dot(ef, Wl, preferred_element_type=jnp.float32)
        hs.append(h.reshape(n, n, 64))
    H = jnp.stack(hs) + bl
    H = jnp.where(H >= 0.0, H, 0.2 * H)
    bias = jnp.where(sel, 0.0, -jnp.inf)
    H = H + bias[:, :, :, None]
    return jnp.max(H, axis=2)


def _body(x_ref, cw_ref, cb_ref, W1_ref, b1_ref, W2_ref, b2_ref,
          Wout_ref, bout_ref, out_ref):
    n = N_PTS
    w_c = x_ref.shape[3]
    xb = x_ref[...].astype(jnp.bfloat16).astype(jnp.float32)
    cw = cw_ref[...].astype(jnp.bfloat16).astype(jnp.float32)
    acc = jnp.zeros((BB, n, w_c), dtype=jnp.float32)
    z = jnp.zeros((BB, n, 1), dtype=jnp.float32)
    for i in range(5):
        s = xb[:, :, i, :]
        sl = jnp.concatenate([z, s[:, :, : w_c - 1]], axis=2)
        sr = jnp.concatenate([s[:, :, 1:], z], axis=2)
        acc = acc + cw[i, 0] * sl
        acc = acc + cw[i, 1] * s
        acc = acc + cw[i, 2] * sr
    y = acc + cb_ref[0, 0]
    pts = jnp.where(y >= 0.0, y, 0.01 * y)

    h1 = _edge_layer(pts, W1_ref[...], b1_ref[...], n, 128)
    h2 = _edge_layer(h1, W2_ref[...], b2_ref[...], n, 64)

    o = jnp.dot(h2.reshape(BB * n, 64), Wout_ref[...],
                preferred_element_type=jnp.float32)
    o = o.reshape(BB, n, 16) + bout_ref[...]
    out_ref[...] = jnp.max(o, axis=1).reshape(1, BB, 16)


def kernel(x, conv_w, conv_b, W1, b1, W2, b2, Wout, bout):
    B, H, W = x.shape
    n = N_PTS
    x4 = x.reshape(B, n, 5, W)
    cw = conv_w.reshape(5, 3)
    cb = conv_b.reshape(1, 1)
    grid = B // BB

    return pl.pallas_call(
        _body,
        grid=(grid,),
        in_specs=[
            pl.BlockSpec((BB, n, 5, W), lambda i: (i, 0, 0, 0)),
            pl.BlockSpec((5, 3), lambda i: (0, 0)),
            pl.BlockSpec(memory_space=pltpu.SMEM),
            pl.BlockSpec((256, 64), lambda i: (0, 0)),
            pl.BlockSpec((1, 64), lambda i: (0, 0)),
            pl.BlockSpec((128, 64), lambda i: (0, 0)),
            pl.BlockSpec((1, 64), lambda i: (0, 0)),
            pl.BlockSpec((64, 16), lambda i: (0, 0)),
            pl.BlockSpec((1, 16), lambda i: (0, 0)),
        ],
        out_specs=pl.BlockSpec((1, BB, 16), lambda i: (i, 0, 0)),
        out_shape=jax.ShapeDtypeStruct((grid, BB, 16), jnp.float32),
    )(x4, cw, cb, W1, b1.reshape(1, 64), W2, b2.reshape(1, 64),
      Wout, bout.reshape(1, 16)).reshape(B, 16)


# speedup vs baseline: 1.0027x; 1.0027x over previous
import jax
import jax.numpy as jnp
from jax.experimental import pallas as pl
from jax.experimental.pallas import tpu as pltpu

K_NN = 20
N_PTS = 40
BB = 8


def _edge_layer(feats, Wl, bl, n, d_in):
    sq = jnp.sum(feats * feats, axis=-1)
    grams = []
    for b in range(BB):
        fb = feats[b]
        grams.append(jax.lax.dot_general(
            fb, fb, (((1,), (1,)), ((), ())),
            preferred_element_type=jnp.float32))
    G = jnp.stack(grams)
    nd = 2.0 * G - sq[:, :, None] - sq[:, None, :]

    a = nd[:, :, :, None]
    c = nd[:, :, None, :]
    ii = jax.lax.broadcasted_iota(jnp.int32, (n, n), 0)
    jj = jax.lax.broadcasted_iota(jnp.int32, (n, n), 1)
    tie = (ii < jj).astype(jnp.float32)[None, None, :, :]
    cmp = (a > c).astype(jnp.float32) + (a == c).astype(jnp.float32) * tie
    rank = jnp.sum(cmp, axis=2)
    sel = rank < float(K_NN)

    ef_c = jnp.broadcast_to(feats[:, :, None, :], (BB, n, n, d_in))
    ef_d = feats[:, None, :, :] - feats[:, :, None, :]
    ef = jnp.concatenate([ef_c, ef_d], axis=-1).reshape(BB * n * n, 2 * d_in)
    h = jnp.dot(ef, Wl, preferred_element_type=jnp.float32)
    H = h.reshape(BB, n, n, 64) + bl
    H = jnp.where(H >= 0.0, H, 0.2 * H)
    bias = jnp.where(sel, 0.0, -jnp.inf)
    H = H + bias[:, :, :, None]
    return jnp.max(H, axis=2)


def _body(x_ref, cw_ref, cb_ref, W1_ref, b1_ref, W2_ref, b2_ref,
          Wout_ref, bout_ref, out_ref):
    n = N_PTS
    w_c = x_ref.shape[3]
    xb = x_ref[...].astype(jnp.bfloat16).astype(jnp.float32)
    cw = cw_ref[...].astype(jnp.bfloat16).astype(jnp.float32)
    acc = jnp.zeros((BB, n, w_c), dtype=jnp.float32)
    z = jnp.zeros((BB, n, 1), dtype=jnp.float32)
    for i in range(5):
        s = xb[:, :, i, :]
        sl = jnp.concatenate([z, s[:, :, : w_c - 1]], axis=2)
        sr = jnp.concatenate([s[:, :, 1:], z], axis=2)
        acc = acc + cw[i, 0] * sl
        acc = acc + cw[i, 1] * s
        acc = acc + cw[i, 2] * sr
    y = acc + cb_ref[0, 0]
    pts = jnp.where(y >= 0.0, y, 0.01 * y)

    h1 = _edge_layer(pts, W1_ref[...], b1_ref[...], n, 128)
    h2 = _edge_layer(h1, W2_ref[...], b2_ref[...], n, 64)

    o = jnp.dot(h2.reshape(BB * n, 64), Wout_ref[...],
                preferred_element_type=jnp.float32)
    o = o.reshape(BB, n, 16) + bout_ref[...]
    out_ref[...] = jnp.max(o, axis=1).reshape(1, BB, 16)


def kernel(x, conv_w, conv_b, W1, b1, W2, b2, Wout, bout):
    B, H, W = x.shape
    n = N_PTS
    x4 = x.reshape(B, n, 5, W)
    cw = conv_w.reshape(5, 3)
    cb = conv_b.reshape(1, 1)
    grid = B // BB

    return pl.pallas_call(
        _body,
        grid=(grid,),
        in_specs=[
            pl.BlockSpec((BB, n, 5, W), lambda i: (i, 0, 0, 0)),
            pl.BlockSpec((5, 3), lambda i: (0, 0)),
            pl.BlockSpec(memory_space=pltpu.SMEM),
            pl.BlockSpec((256, 64), lambda i: (0, 0)),
            pl.BlockSpec((1, 64), lambda i: (0, 0)),
            pl.BlockSpec((128, 64), lambda i: (0, 0)),
            pl.BlockSpec((1, 64), lambda i: (0, 0)),
            pl.BlockSpec((64, 16), lambda i: (0, 0)),
            pl.BlockSpec((1, 16), lambda i: (0, 0)),
        ],
        out_specs=pl.BlockSpec((1, BB, 16), lambda i: (i, 0, 0)),
        out_shape=jax.ShapeDtypeStruct((grid, BB, 16), jnp.float32),
    )(x4, cw, cb, W1, b1.reshape(1, 64), W2, b2.reshape(1, 64),
      Wout, bout.reshape(1, 16)).reshape(B, 16)


# speedup vs baseline: 1.0071x; 1.0043x over previous
import jax
import jax.numpy as jnp
from jax.experimental import pallas as pl
from jax.experimental.pallas import tpu as pltpu

K_NN = 20
N_PTS = 40
BB = 8


def _edge_layer(feats, Wl, bl, n, d_in):
    sq = jnp.sum(feats * feats, axis=-1)
    grams = []
    for b in range(BB):
        fb = feats[b]
        grams.append(jax.lax.dot_general(
            fb, fb, (((1,), (1,)), ((), ())),
            preferred_element_type=jnp.float32))
    G = jnp.stack(grams)
    nd = 2.0 * G - sq[:, :, None] - sq[:, None, :]

    a = nd[:, :, :, None]
    c = nd[:, :, None, :]
    ii = jax.lax.broadcasted_iota(jnp.int32, (n, n), 0)
    jj = jax.lax.broadcasted_iota(jnp.int32, (n, n), 1)
    tie = (ii < jj).astype(jnp.float32)[None, None, :, :]
    cmp = (a > c).astype(jnp.float32) + (a == c).astype(jnp.float32) * tie
    rank = jnp.sum(cmp, axis=2)
    sel = rank < float(K_NN)

    hs = []
    for b in range(BB):
        fc = feats[b]
        ef_c = jnp.broadcast_to(fc[:, None, :], (n, n, d_in))
        ef_d = fc[None, :, :] - fc[:, None, :]
        ef = jnp.concatenate([ef_c, ef_d], axis=-1).reshape(n * n, 2 * d_in)
        h = jnp.dot(ef, Wl, preferred_element_type=jnp.float32)
        hs.append(h.reshape(n, n, 64))
    H = jnp.stack(hs) + bl
    H = jnp.where(H >= 0.0, H, 0.2 * H)
    bias = jnp.where(sel, 0.0, -jnp.inf)
    H = H + bias[:, :, :, None]
    return jnp.max(H, axis=2)


def _body(x_ref, cw_ref, cb_ref, W1_ref, b1_ref, W2_ref, b2_ref,
          Wout_ref, bout_ref, out_ref):
    n = N_PTS
    w_c = x_ref.shape[3]
    xb = x_ref[...].astype(jnp.bfloat16).astype(jnp.float32)
    cw = cw_ref[...].astype(jnp.bfloat16).astype(jnp.float32)
    acc = jnp.zeros((BB, n, w_c), dtype=jnp.float32)
    z = jnp.zeros((BB, n, 1), dtype=jnp.float32)
    for i in range(5):
        s = xb[:, :, i, :]
        sl = jnp.concatenate([z, s[:, :, : w_c - 1]], axis=2)
        sr = jnp.concatenate([s[:, :, 1:], z], axis=2)
        acc = acc + cw[i, 0] * sl
        acc = acc + cw[i, 1] * s
        acc = acc + cw[i, 2] * sr
    y = acc + cb_ref[0, 0]
    pts = jnp.where(y >= 0.0, y, 0.01 * y)

    h1 = _edge_layer(pts, W1_ref[...], b1_ref[...], n, 128)
    h2 = _edge_layer(h1, W2_ref[...], b2_ref[...], n, 64)

    o = jnp.dot(h2.reshape(BB * n, 64), Wout_ref[...],
                preferred_element_type=jnp.float32)
    o = o.reshape(BB, n, 16) + bout_ref[...]
    out_ref[...] = jnp.max(o, axis=1).reshape(1, BB, 16)


def kernel(x, conv_w, conv_b, W1, b1, W2, b2, Wout, bout):
    B, H, W = x.shape
    n = N_PTS
    x4 = x.reshape(B, n, 5, W)
    cw = conv_w.reshape(5, 3)
    cb = conv_b.reshape(1, 1)
    grid = B // BB

    return pl.pallas_call(
        _body,
        grid=(grid,),
        in_specs=[
            pl.BlockSpec((BB, n, 5, W), lambda i: (i, 0, 0, 0)),
            pl.BlockSpec((5, 3), lambda i: (0, 0)),
            pl.BlockSpec(memory_space=pltpu.SMEM),
            pl.BlockSpec((256, 64), lambda i: (0, 0)),
            pl.BlockSpec((1, 64), lambda i: (0, 0)),
            pl.BlockSpec((128, 64), lambda i: (0, 0)),
            pl.BlockSpec((1, 64), lambda i: (0, 0)),
            pl.BlockSpec((64, 16), lambda i: (0, 0)),
            pl.BlockSpec((1, 16), lambda i: (0, 0)),
        ],
        out_specs=pl.BlockSpec((1, BB, 16), lambda i: (i, 0, 0)),
        out_shape=jax.ShapeDtypeStruct((grid, BB, 16), jnp.float32),
    )(x4, cw, cb, W1, b1.reshape(1, 64), W2, b2.reshape(1, 64),
      Wout, bout.reshape(1, 16)).reshape(B, 16)
